# baseline (speedup 1.0000x reference)
import functools

import jax
import jax.numpy as jnp
from jax.experimental import pallas as pl
from jax.experimental.pallas import tpu as pltpu

DIM = 768
D_STATE = 16
D_INNER = 1536
NUM_EXPERTS = 8
D_CONV = 4
DT_RANK = 48
HIDDEN = 1536
L = 2048

TB = 512
TBM = 1024


def _bf(a):
    return a.astype(jnp.bfloat16).astype(jnp.float32)


def _moe_body(x_ref, gw_ref, gb_ref, w1_ref, b1_ref, w2_ref, b2_ref, o_ref):
    e = pl.program_id(1)
    x = x_ref[...]
    logits = jnp.dot(x, gw_ref[...]) + gb_ref[...]
    m = jnp.max(logits, axis=-1, keepdims=True)
    eu = jnp.exp(logits - m)
    probs = eu / jnp.sum(eu, axis=-1, keepdims=True)
    pm = jnp.max(probs, axis=-1, keepdims=True)
    lane = jax.lax.broadcasted_iota(jnp.int32, (TBM, NUM_EXPERTS), 1)
    cand = jnp.where(probs == pm, lane, NUM_EXPERTS)
    idx = jnp.min(cand, axis=-1, keepdims=True)
    g = jnp.where(idx == e, pm, 0.0)

    h = jax.nn.gelu(jnp.dot(x.astype(jnp.bfloat16), w1_ref[0],
                            preferred_element_type=jnp.float32) + b1_ref[0])
    eo = jnp.dot(h.astype(jnp.bfloat16), w2_ref[0],
                 preferred_element_type=jnp.float32) + b2_ref[0]
    contrib = _bf(g) * _bf(eo)

    @pl.when(e == 0)
    def _():
        o_ref[...] = contrib

    @pl.when(e != 0)
    def _():
        o_ref[...] += contrib


def _moe(x2, p):
    out = pl.pallas_call(
        _moe_body,
        grid=(L // TBM, NUM_EXPERTS),
        in_specs=[
            pl.BlockSpec((TBM, DIM), lambda t, e: (t, 0)),
            pl.BlockSpec((DIM, NUM_EXPERTS), lambda t, e: (0, 0)),
            pl.BlockSpec((1, NUM_EXPERTS), lambda t, e: (0, 0)),
            pl.BlockSpec((1, DIM, HIDDEN), lambda t, e: (e, 0, 0)),
            pl.BlockSpec((1, 1, HIDDEN), lambda t, e: (e, 0, 0)),
            pl.BlockSpec((1, HIDDEN, DIM), lambda t, e: (e, 0, 0)),
            pl.BlockSpec((1, 1, DIM), lambda t, e: (e, 0, 0)),
        ],
        out_specs=pl.BlockSpec((TBM, DIM), lambda t, e: (t, 0)),
        out_shape=jax.ShapeDtypeStruct((L, DIM), jnp.float32),
        compiler_params=pltpu.CompilerParams(
            dimension_semantics=("arbitrary", "arbitrary")),
    )(x2, p['gate_w'], p['gate_b'][None, :],
      p['w1'].astype(jnp.bfloat16), p['b1'][:, None, :],
      p['w2'].astype(jnp.bfloat16), p['b2'][:, None, :])
    return out


def _pre_body(xpad_ref, w1_ref, w2_ref, cw_ref, cb_ref, xp_ref, dtw_ref,
              dtb_ref, xc_ref, delta_ref, b_ref, c_ref, res_ref):
    t = pl.program_id(0)
    xe = xpad_ref[pl.ds(t * TB, TB + 8)]
    xpe = jnp.dot(xe, w1_ref[...])
    cw = cw_ref[...]
    acc = xpe[8:] * cw[:, D_CONV - 1][None, :]
    for k in range(1, D_CONV):
        acc = acc + xpe[8 - k:8 - k + TB] * cw[:, D_CONV - 1 - k][None, :]
    acc = acc + cb_ref[...]
    xc = acc * jax.nn.sigmoid(acc)
    xc_ref[...] = xc
    res_ref[...] = jnp.dot(xe[8:], w2_ref[...])
    x_dbl = jnp.dot(xc, xp_ref[...])
    dt = x_dbl[:, :DT_RANK]
    b_ref[...] = x_dbl[:, DT_RANK:DT_RANK + D_STATE]
    c_ref[...] = x_dbl[:, DT_RANK + D_STATE:]
    delta_ref[...] = jax.nn.softplus(
        jnp.dot(dt, dtw_ref[...]) + dtb_ref[...])


def _scan_body(alogt_ref, d_ref, u_ref, b_ref, c_ref, ys_ref,
               h_ref, da_ref, dbu_ref, h3_ref, *, chunk):
    g = pl.program_id(1)
    at = -jnp.exp(alogt_ref[...])
    d = d_ref[...]
    u = u_ref[...]
    da_ref[...] = jnp.exp(d[:, None, :] * at[None, :, :])
    dbu_ref[...] = (d * u)[:, None, :] * b_ref[...][:, :, None]

    @pl.when(g == 0)
    def _():
        h_ref[...] = jnp.zeros_like(h_ref)

    def step(t, _):
        h = da_ref[t] * h_ref[...] + dbu_ref[t]
        h_ref[...] = h
        h3_ref[t] = h
        return 0

    jax.lax.fori_loop(0, chunk, step, 0, unroll=2)
    ys_ref[...] = jnp.sum(h3_ref[...] * c_ref[...][:, :, None], axis=1)


def _post_body(ys_ref, xc_ref, dvec_ref, res_ref, wout_ref, xres_ref, o_ref):
    y = ys_ref[...] + xc_ref[...] * dvec_ref[...]
    res = res_ref[...]
    y = y * (res * jax.nn.sigmoid(res))
    o_ref[...] = jnp.dot(y, wout_ref[...]) + xres_ref[...]


def _mamba(x2, p):
    w_in = p['in_proj']
    xpad = jnp.pad(x2, ((8, 0), (0, 0)))

    xc, delta, bm, cm, res = pl.pallas_call(
        _pre_body,
        grid=(L // TB,),
        in_specs=[
            pl.BlockSpec((L + 8, DIM), lambda t: (0, 0)),
            pl.BlockSpec((DIM, D_INNER), lambda t: (0, 0)),
            pl.BlockSpec((DIM, D_INNER), lambda t: (0, 0)),
            pl.BlockSpec((D_INNER, D_CONV), lambda t: (0, 0)),
            pl.BlockSpec((1, D_INNER), lambda t: (0, 0)),
            pl.BlockSpec((D_INNER, DT_RANK + 2 * D_STATE), lambda t: (0, 0)),
            pl.BlockSpec((DT_RANK, D_INNER), lambda t: (0, 0)),
            pl.BlockSpec((1, D_INNER), lambda t: (0, 0)),
        ],
        out_specs=[
            pl.BlockSpec((TB, D_INNER), lambda t: (t, 0)),
            pl.BlockSpec((TB, D_INNER), lambda t: (t, 0)),
            pl.BlockSpec((TB, D_STATE), lambda t: (t, 0)),
            pl.BlockSpec((TB, D_STATE), lambda t: (t, 0)),
            pl.BlockSpec((TB, D_INNER), lambda t: (t, 0)),
        ],
        out_shape=[
            jax.ShapeDtypeStruct((L, D_INNER), jnp.float32),
            jax.ShapeDtypeStruct((L, D_INNER), jnp.float32),
            jax.ShapeDtypeStruct((L, D_STATE), jnp.float32),
            jax.ShapeDtypeStruct((L, D_STATE), jnp.float32),
            jax.ShapeDtypeStruct((L, D_INNER), jnp.float32),
        ],
    )(xpad, w_in[:, :D_INNER], w_in[:, D_INNER:], p['conv_w'],
      p['conv_b'][None, :], p['x_proj'], p['dt_proj_w'],
      p['dt_proj_b'][None, :])

    chunk = 64
    nchunk = L // chunk
    DH = D_INNER // 2
    ys = pl.pallas_call(
        functools.partial(_scan_body, chunk=chunk),
        grid=(2, nchunk),
        in_specs=[
            pl.BlockSpec((D_STATE, DH), lambda c, g: (0, c)),
            pl.BlockSpec((chunk, DH), lambda c, g: (g, c)),
            pl.BlockSpec((chunk, DH), lambda c, g: (g, c)),
            pl.BlockSpec((chunk, D_STATE), lambda c, g: (g, 0)),
            pl.BlockSpec((chunk, D_STATE), lambda c, g: (g, 0)),
        ],
        out_specs=pl.BlockSpec((chunk, DH), lambda c, g: (g, c)),
        out_shape=jax.ShapeDtypeStruct((L, D_INNER), jnp.float32),
        scratch_shapes=[
            pltpu.VMEM((D_STATE, DH), jnp.float32),
            pltpu.VMEM((chunk, D_STATE, DH), jnp.float32),
            pltpu.VMEM((chunk, D_STATE, DH), jnp.float32),
            pltpu.VMEM((chunk, D_STATE, DH), jnp.float32),
        ],
        compiler_params=pltpu.CompilerParams(
            dimension_semantics=("parallel", "arbitrary")),
    )(p['A_log'].T, delta, xc, bm, cm)

    out = pl.pallas_call(
        _post_body,
        grid=(L // TB,),
        in_specs=[
            pl.BlockSpec((TB, D_INNER), lambda r: (r, 0)),
            pl.BlockSpec((TB, D_INNER), lambda r: (r, 0)),
            pl.BlockSpec((1, D_INNER), lambda r: (0, 0)),
            pl.BlockSpec((TB, D_INNER), lambda r: (r, 0)),
            pl.BlockSpec((D_INNER, DIM), lambda r: (0, 0)),
            pl.BlockSpec((TB, DIM), lambda r: (r, 0)),
        ],
        out_specs=pl.BlockSpec((TB, DIM), lambda r: (r, 0)),
        out_shape=jax.ShapeDtypeStruct((L, DIM), jnp.float32),
    )(ys, xc, p['D'][None, :], res, p['out_proj'], x2)
    return out


def kernel(x, params):
    x2 = x[0]
    for lp in params:
        x2 = _moe(x2, lp['moe'])
        x2 = _mamba(x2, lp['mamba'])
        x2 = _moe(x2, lp['moe'])
    return x2[None]


# speedup vs baseline: 1.0048x; 1.0048x over previous
import functools

import jax
import jax.numpy as jnp
from jax.experimental import pallas as pl
from jax.experimental.pallas import tpu as pltpu

DIM = 768
D_STATE = 16
D_INNER = 1536
NUM_EXPERTS = 8
D_CONV = 4
DT_RANK = 48
HIDDEN = 1536
L = 2048

TB = 512
TBM = 512


def _bf(a):
    return a.astype(jnp.bfloat16).astype(jnp.float32)


def _moe_body(x_ref, gw_ref, gb_ref, w1_ref, b1_ref, w2_ref, b2_ref, o_ref):
    e = pl.program_id(1)
    x = x_ref[...]
    logits = jnp.dot(x, gw_ref[...]) + gb_ref[...]
    m = jnp.max(logits, axis=-1, keepdims=True)
    eu = jnp.exp(logits - m)
    probs = eu / jnp.sum(eu, axis=-1, keepdims=True)
    pm = jnp.max(probs, axis=-1, keepdims=True)
    lane = jax.lax.broadcasted_iota(jnp.int32, (TBM, NUM_EXPERTS), 1)
    cand = jnp.where(probs == pm, lane, NUM_EXPERTS)
    idx = jnp.min(cand, axis=-1, keepdims=True)
    g = jnp.where(idx == e, pm, 0.0)

    h = jax.nn.gelu(jnp.dot(x.astype(jnp.bfloat16), w1_ref[0],
                            preferred_element_type=jnp.float32) + b1_ref[0])
    eo = jnp.dot(h.astype(jnp.bfloat16), w2_ref[0],
                 preferred_element_type=jnp.float32) + b2_ref[0]
    contrib = _bf(g) * _bf(eo)

    @pl.when(e == 0)
    def _():
        o_ref[...] = contrib

    @pl.when(e != 0)
    def _():
        o_ref[...] += contrib


def _moe(x2, p):
    out = pl.pallas_call(
        _moe_body,
        grid=(L // TBM, NUM_EXPERTS),
        in_specs=[
            pl.BlockSpec((TBM, DIM), lambda t, e: (t, 0)),
            pl.BlockSpec((DIM, NUM_EXPERTS), lambda t, e: (0, 0)),
            pl.BlockSpec((1, NUM_EXPERTS), lambda t, e: (0, 0)),
            pl.BlockSpec((1, DIM, HIDDEN), lambda t, e: (e, 0, 0)),
            pl.BlockSpec((1, 1, HIDDEN), lambda t, e: (e, 0, 0)),
            pl.BlockSpec((1, HIDDEN, DIM), lambda t, e: (e, 0, 0)),
            pl.BlockSpec((1, 1, DIM), lambda t, e: (e, 0, 0)),
        ],
        out_specs=pl.BlockSpec((TBM, DIM), lambda t, e: (t, 0)),
        out_shape=jax.ShapeDtypeStruct((L, DIM), jnp.float32),
        compiler_params=pltpu.CompilerParams(
            dimension_semantics=("arbitrary", "arbitrary")),
    )(x2, p['gate_w'], p['gate_b'][None, :],
      p['w1'].astype(jnp.bfloat16), p['b1'][:, None, :],
      p['w2'].astype(jnp.bfloat16), p['b2'][:, None, :])
    return out


def _pre_body(xpad_ref, w1_ref, w2_ref, cw_ref, cb_ref, xp_ref, dtw_ref,
              dtb_ref, xc_ref, delta_ref, b_ref, c_ref, res_ref):
    t = pl.program_id(0)
    xe = xpad_ref[pl.ds(t * TB, TB + 8)]
    xpe = jnp.dot(xe, w1_ref[...])
    cw = cw_ref[...]
    acc = xpe[8:] * cw[:, D_CONV - 1][None, :]
    for k in range(1, D_CONV):
        acc = acc + xpe[8 - k:8 - k + TB] * cw[:, D_CONV - 1 - k][None, :]
    acc = acc + cb_ref[...]
    xc = acc * jax.nn.sigmoid(acc)
    xc_ref[...] = xc
    res_ref[...] = jnp.dot(xe[8:], w2_ref[...])
    x_dbl = jnp.dot(xc, xp_ref[...])
    dt = x_dbl[:, :DT_RANK]
    b_ref[...] = x_dbl[:, DT_RANK:DT_RANK + D_STATE]
    c_ref[...] = x_dbl[:, DT_RANK + D_STATE:]
    delta_ref[...] = jax.nn.softplus(
        jnp.dot(dt, dtw_ref[...]) + dtb_ref[...])


def _scan_body(alogt_ref, d_ref, u_ref, b_ref, c_ref, ys_ref,
               h_ref, da_ref, dbu_ref, h3_ref, *, chunk):
    g = pl.program_id(1)
    at = -jnp.exp(alogt_ref[...])
    d = d_ref[...]
    u = u_ref[...]
    da_ref[...] = jnp.exp(d[:, None, :] * at[None, :, :])
    dbu_ref[...] = (d * u)[:, None, :] * b_ref[...][:, :, None]

    @pl.when(g == 0)
    def _():
        h_ref[...] = jnp.zeros_like(h_ref)

    def step(t, _):
        h = da_ref[t] * h_ref[...] + dbu_ref[t]
        h_ref[...] = h
        h3_ref[t] = h
        return 0

    jax.lax.fori_loop(0, chunk, step, 0, unroll=2)
    ys_ref[...] = jnp.sum(h3_ref[...] * c_ref[...][:, :, None], axis=1)


def _post_body(ys_ref, xc_ref, dvec_ref, res_ref, wout_ref, xres_ref, o_ref):
    y = ys_ref[...] + xc_ref[...] * dvec_ref[...]
    res = res_ref[...]
    y = y * (res * jax.nn.sigmoid(res))
    o_ref[...] = jnp.dot(y, wout_ref[...]) + xres_ref[...]


def _mamba(x2, p):
    w_in = p['in_proj']
    xpad = jnp.pad(x2, ((8, 0), (0, 0)))

    xc, delta, bm, cm, res = pl.pallas_call(
        _pre_body,
        grid=(L // TB,),
        in_specs=[
            pl.BlockSpec((L + 8, DIM), lambda t: (0, 0)),
            pl.BlockSpec((DIM, D_INNER), lambda t: (0, 0)),
            pl.BlockSpec((DIM, D_INNER), lambda t: (0, 0)),
            pl.BlockSpec((D_INNER, D_CONV), lambda t: (0, 0)),
            pl.BlockSpec((1, D_INNER), lambda t: (0, 0)),
            pl.BlockSpec((D_INNER, DT_RANK + 2 * D_STATE), lambda t: (0, 0)),
            pl.BlockSpec((DT_RANK, D_INNER), lambda t: (0, 0)),
            pl.BlockSpec((1, D_INNER), lambda t: (0, 0)),
        ],
        out_specs=[
            pl.BlockSpec((TB, D_INNER), lambda t: (t, 0)),
            pl.BlockSpec((TB, D_INNER), lambda t: (t, 0)),
            pl.BlockSpec((TB, D_STATE), lambda t: (t, 0)),
            pl.BlockSpec((TB, D_STATE), lambda t: (t, 0)),
            pl.BlockSpec((TB, D_INNER), lambda t: (t, 0)),
        ],
        out_shape=[
            jax.ShapeDtypeStruct((L, D_INNER), jnp.float32),
            jax.ShapeDtypeStruct((L, D_INNER), jnp.float32),
            jax.ShapeDtypeStruct((L, D_STATE), jnp.float32),
            jax.ShapeDtypeStruct((L, D_STATE), jnp.float32),
            jax.ShapeDtypeStruct((L, D_INNER), jnp.float32),
        ],
    )(xpad, w_in[:, :D_INNER], w_in[:, D_INNER:], p['conv_w'],
      p['conv_b'][None, :], p['x_proj'], p['dt_proj_w'],
      p['dt_proj_b'][None, :])

    chunk = 64
    nchunk = L // chunk
    DH = D_INNER // 2
    ys = pl.pallas_call(
        functools.partial(_scan_body, chunk=chunk),
        grid=(2, nchunk),
        in_specs=[
            pl.BlockSpec((D_STATE, DH), lambda c, g: (0, c)),
            pl.BlockSpec((chunk, DH), lambda c, g: (g, c)),
            pl.BlockSpec((chunk, DH), lambda c, g: (g, c)),
            pl.BlockSpec((chunk, D_STATE), lambda c, g: (g, 0)),
            pl.BlockSpec((chunk, D_STATE), lambda c, g: (g, 0)),
        ],
        out_specs=pl.BlockSpec((chunk, DH), lambda c, g: (g, c)),
        out_shape=jax.ShapeDtypeStruct((L, D_INNER), jnp.float32),
        scratch_shapes=[
            pltpu.VMEM((D_STATE, DH), jnp.float32),
            pltpu.VMEM((chunk, D_STATE, DH), jnp.float32),
            pltpu.VMEM((chunk, D_STATE, DH), jnp.float32),
            pltpu.VMEM((chunk, D_STATE, DH), jnp.float32),
        ],
        compiler_params=pltpu.CompilerParams(
            dimension_semantics=("parallel", "arbitrary")),
    )(p['A_log'].T, delta, xc, bm, cm)

    out = pl.pallas_call(
        _post_body,
        grid=(L // TB,),
        in_specs=[
            pl.BlockSpec((TB, D_INNER), lambda r: (r, 0)),
            pl.BlockSpec((TB, D_INNER), lambda r: (r, 0)),
            pl.BlockSpec((1, D_INNER), lambda r: (0, 0)),
            pl.BlockSpec((TB, D_INNER), lambda r: (r, 0)),
            pl.BlockSpec((D_INNER, DIM), lambda r: (0, 0)),
            pl.BlockSpec((TB, DIM), lambda r: (r, 0)),
        ],
        out_specs=pl.BlockSpec((TB, DIM), lambda r: (r, 0)),
        out_shape=jax.ShapeDtypeStruct((L, DIM), jnp.float32),
    )(ys, xc, p['D'][None, :], res, p['out_proj'], x2)
    return out


def kernel(x, params):
    x2 = x[0]
    for lp in params:
        x2 = _moe(x2, lp['moe'])
        x2 = _mamba(x2, lp['mamba'])
        x2 = _moe(x2, lp['moe'])
    return x2[None]


# speedup vs baseline: 1.0426x; 1.0376x over previous
import functools

import jax
import jax.numpy as jnp
from jax.experimental import pallas as pl
from jax.experimental.pallas import tpu as pltpu

DIM = 768
D_STATE = 16
D_INNER = 1536
NUM_EXPERTS = 8
D_CONV = 4
DT_RANK = 48
HIDDEN = 1536
L = 2048

TB = 512


def _bf(a):
    return a.astype(jnp.bfloat16).astype(jnp.float32)


def _moe_body(x_ref, gw_ref, gb_ref, w1_ref, b1_ref, w2_ref, b2_ref, o_ref):
    e = pl.program_id(1)
    x = x_ref[...]
    logits = jnp.dot(x, gw_ref[...]) + gb_ref[...]
    m = jnp.max(logits, axis=-1, keepdims=True)
    eu = jnp.exp(logits - m)
    probs = eu / jnp.sum(eu, axis=-1, keepdims=True)
    pm = jnp.max(probs, axis=-1, keepdims=True)
    lane = jax.lax.broadcasted_iota(jnp.int32, (TB, NUM_EXPERTS), 1)
    cand = jnp.where(probs == pm, lane, NUM_EXPERTS)
    idx = jnp.min(cand, axis=-1, keepdims=True)
    g = jnp.where(idx == e, pm, 0.0)

    h = jax.nn.gelu(jnp.dot(x, w1_ref[0]) + b1_ref[0])
    eo = jnp.dot(h, w2_ref[0]) + b2_ref[0]
    contrib = _bf(g) * _bf(eo)

    @pl.when(e == 0)
    def _():
        o_ref[...] = contrib

    @pl.when(e != 0)
    def _():
        o_ref[...] += contrib


def _moe(x2, p):
    out = pl.pallas_call(
        _moe_body,
        grid=(L // TB, NUM_EXPERTS),
        in_specs=[
            pl.BlockSpec((TB, DIM), lambda t, e: (t, 0)),
            pl.BlockSpec((DIM, NUM_EXPERTS), lambda t, e: (0, 0)),
            pl.BlockSpec((1, NUM_EXPERTS), lambda t, e: (0, 0)),
            pl.BlockSpec((1, DIM, HIDDEN), lambda t, e: (e, 0, 0)),
            pl.BlockSpec((1, 1, HIDDEN), lambda t, e: (e, 0, 0)),
            pl.BlockSpec((1, HIDDEN, DIM), lambda t, e: (e, 0, 0)),
            pl.BlockSpec((1, 1, DIM), lambda t, e: (e, 0, 0)),
        ],
        out_specs=pl.BlockSpec((TB, DIM), lambda t, e: (t, 0)),
        out_shape=jax.ShapeDtypeStruct((L, DIM), jnp.float32),
        compiler_params=pltpu.CompilerParams(
            dimension_semantics=("arbitrary", "arbitrary")),
    )(x2, p['gate_w'], p['gate_b'][None, :], p['w1'], p['b1'][:, None, :],
      p['w2'], p['b2'][:, None, :])
    return out


def _pre_body(xpad_ref, w1_ref, w2_ref, cw_ref, cb_ref, xp_ref, dtw_ref,
              dtb_ref, xc_ref, delta_ref, b_ref, c_ref, res_ref):
    t = pl.program_id(0)
    xe = xpad_ref[pl.ds(t * TB, TB + 8)]
    xpe = jnp.dot(xe, w1_ref[...])
    cw = cw_ref[...]
    acc = xpe[8:] * cw[:, D_CONV - 1][None, :]
    for k in range(1, D_CONV):
        acc = acc + xpe[8 - k:8 - k + TB] * cw[:, D_CONV - 1 - k][None, :]
    acc = acc + cb_ref[...]
    xc = acc * jax.nn.sigmoid(acc)
    xc_ref[...] = xc
    res_ref[...] = jnp.dot(xe[8:], w2_ref[...])
    x_dbl = jnp.dot(xc, xp_ref[...])
    dt = x_dbl[:, :DT_RANK]
    b_ref[...] = x_dbl[:, DT_RANK:DT_RANK + D_STATE]
    c_ref[...] = x_dbl[:, DT_RANK + D_STATE:]
    delta_ref[...] = jax.nn.softplus(
        jnp.dot(dt, dtw_ref[...]) + dtb_ref[...])


def _scan_body(alogt_ref, d_ref, u_ref, b_ref, c_ref, ys_ref,
               h_ref, da_ref, dbu_ref, h3_ref, *, chunk):
    g = pl.program_id(1)
    at = -jnp.exp(alogt_ref[...])
    d = d_ref[...]
    u = u_ref[...]
    da_ref[...] = jnp.exp(d[:, None, :] * at[None, :, :])
    dbu_ref[...] = (d * u)[:, None, :] * b_ref[...][:, :, None]

    @pl.when(g == 0)
    def _():
        h_ref[...] = jnp.zeros_like(h_ref)

    def step(t, _):
        h = da_ref[t] * h_ref[...] + dbu_ref[t]
        h_ref[...] = h
        h3_ref[t] = h
        return 0

    jax.lax.fori_loop(0, chunk, step, 0, unroll=2)
    ys_ref[...] = jnp.sum(h3_ref[...] * c_ref[...][:, :, None], axis=1)


def _post_body(ys_ref, xc_ref, dvec_ref, res_ref, wout_ref, xres_ref, o_ref):
    y = ys_ref[...] + xc_ref[...] * dvec_ref[...]
    res = res_ref[...]
    y = y * (res * jax.nn.sigmoid(res))
    o_ref[...] = jnp.dot(y, wout_ref[...]) + xres_ref[...]


def _mamba(x2, p):
    w_in = p['in_proj']
    xpad = jnp.pad(x2, ((8, 0), (0, 0)))

    xc, delta, bm, cm, res = pl.pallas_call(
        _pre_body,
        grid=(L // TB,),
        in_specs=[
            pl.BlockSpec((L + 8, DIM), lambda t: (0, 0)),
            pl.BlockSpec((DIM, D_INNER), lambda t: (0, 0)),
            pl.BlockSpec((DIM, D_INNER), lambda t: (0, 0)),
            pl.BlockSpec((D_INNER, D_CONV), lambda t: (0, 0)),
            pl.BlockSpec((1, D_INNER), lambda t: (0, 0)),
            pl.BlockSpec((D_INNER, DT_RANK + 2 * D_STATE), lambda t: (0, 0)),
            pl.BlockSpec((DT_RANK, D_INNER), lambda t: (0, 0)),
            pl.BlockSpec((1, D_INNER), lambda t: (0, 0)),
        ],
        out_specs=[
            pl.BlockSpec((TB, D_INNER), lambda t: (t, 0)),
            pl.BlockSpec((TB, D_INNER), lambda t: (t, 0)),
            pl.BlockSpec((TB, D_STATE), lambda t: (t, 0)),
            pl.BlockSpec((TB, D_STATE), lambda t: (t, 0)),
            pl.BlockSpec((TB, D_INNER), lambda t: (t, 0)),
        ],
        out_shape=[
            jax.ShapeDtypeStruct((L, D_INNER), jnp.float32),
            jax.ShapeDtypeStruct((L, D_INNER), jnp.float32),
            jax.ShapeDtypeStruct((L, D_STATE), jnp.float32),
            jax.ShapeDtypeStruct((L, D_STATE), jnp.float32),
            jax.ShapeDtypeStruct((L, D_INNER), jnp.float32),
        ],
    )(xpad, w_in[:, :D_INNER], w_in[:, D_INNER:], p['conv_w'],
      p['conv_b'][None, :], p['x_proj'], p['dt_proj_w'],
      p['dt_proj_b'][None, :])

    chunk = 64
    nchunk = L // chunk
    DH = D_INNER // 2
    ys = pl.pallas_call(
        functools.partial(_scan_body, chunk=chunk),
        grid=(2, nchunk),
        in_specs=[
            pl.BlockSpec((D_STATE, DH), lambda c, g: (0, c)),
            pl.BlockSpec((chunk, DH), lambda c, g: (g, c)),
            pl.BlockSpec((chunk, DH), lambda c, g: (g, c)),
            pl.BlockSpec((chunk, D_STATE), lambda c, g: (g, 0)),
            pl.BlockSpec((chunk, D_STATE), lambda c, g: (g, 0)),
        ],
        out_specs=pl.BlockSpec((chunk, DH), lambda c, g: (g, c)),
        out_shape=jax.ShapeDtypeStruct((L, D_INNER), jnp.float32),
        scratch_shapes=[
            pltpu.VMEM((D_STATE, DH), jnp.float32),
            pltpu.VMEM((chunk, D_STATE, DH), jnp.float32),
            pltpu.VMEM((chunk, D_STATE, DH), jnp.float32),
            pltpu.VMEM((chunk, D_STATE, DH), jnp.float32),
        ],
        compiler_params=pltpu.CompilerParams(
            dimension_semantics=("parallel", "arbitrary")),
    )(p['A_log'].T, delta, xc, bm, cm)

    out = pl.pallas_call(
        _post_body,
        grid=(L // TB,),
        in_specs=[
            pl.BlockSpec((TB, D_INNER), lambda r: (r, 0)),
            pl.BlockSpec((TB, D_INNER), lambda r: (r, 0)),
            pl.BlockSpec((1, D_INNER), lambda r: (0, 0)),
            pl.BlockSpec((TB, D_INNER), lambda r: (r, 0)),
            pl.BlockSpec((D_INNER, DIM), lambda r: (0, 0)),
            pl.BlockSpec((TB, DIM), lambda r: (r, 0)),
        ],
        out_specs=pl.BlockSpec((TB, DIM), lambda r: (r, 0)),
        out_shape=jax.ShapeDtypeStruct((L, DIM), jnp.float32),
    )(ys, xc, p['D'][None, :], res, p['out_proj'], x2)
    return out


def kernel(x, params):
    x2 = x[0]
    for lp in params:
        x2 = _moe(x2, lp['moe'])
        x2 = _mamba(x2, lp['mamba'])
        x2 = _moe(x2, lp['moe'])
    return x2[None]
